# baseline (speedup 1.0000x reference)
import functools

import jax
import jax.numpy as jnp
from jax import lax
from jax.experimental import pallas as pl
from jax.experimental.pallas import tpu as pltpu
from jax.experimental.pallas import tpu_sc as plsc

_T = 4096
_D = 1024
_NC = 2
_NS = 16
_NW = _NC * _NS
_RPW = _T // _NW
_CH = 32
_NCHUNK = _RPW // _CH


def _make_sc_copy():
    mesh = plsc.VectorSubcoreMesh(core_axis_name="c", subcore_axis_name="s")

    @functools.partial(
        pl.kernel,
        mesh=mesh,
        out_type=jax.ShapeDtypeStruct((_T, _D), jnp.float32),
        scratch_types=[
            pltpu.VMEM((_CH, _D), jnp.float32),
            pltpu.VMEM((_CH, _D), jnp.float32),
            pltpu.SemaphoreType.DMA,
            pltpu.SemaphoreType.DMA,
        ],
    )
    def sc_copy(table_hbm, out_hbm, buf0, buf1, sem0, sem1):
        wid = lax.axis_index("s") * _NC + lax.axis_index("c")
        base = wid * _RPW
        bufs = (buf0, buf1)
        sems = (sem0, sem1)
        copies = [None] * _NCHUNK
        copies[0] = pltpu.async_copy(table_hbm.at[pl.ds(base, _CH)], bufs[0], sems[0])
        for k in range(_NCHUNK):
            if k + 1 < _NCHUNK:
                copies[k + 1] = pltpu.async_copy(
                    table_hbm.at[pl.ds(base + (k + 1) * _CH, _CH)],
                    bufs[(k + 1) % 2],
                    sems[(k + 1) % 2],
                )
            copies[k].wait()
            pltpu.sync_copy(bufs[k % 2], out_hbm.at[pl.ds(base + k * _CH, _CH)])

    return sc_copy


_sc_copy = _make_sc_copy()


def kernel(x, pe_table):
    del x
    out = _sc_copy(pe_table)
    return out[None]
